# baseline (speedup 1.0000x reference)
import functools

import jax
import jax.numpy as jnp
from jax import lax
from jax.experimental import pallas as pl
from jax.experimental.pallas import tpu as pltpu
from jax.experimental.pallas import tpu_sc as plsc

_NC = 2
_NS = 16
_W = _NC * _NS
_K = 128
_R = 512


def _deg_kernel(npad, c_chunks, w=128):
  rpt = npad // _NS
  mesh = plsc.VectorSubcoreMesh(core_axis_name="c", subcore_axis_name="s")

  nb = 2
  assert c_chunks % nb == 0

  def body(dstw, zeros_hbm, ones_hbm, out_hbm, didx_all, ones_v, acc, *sems):
    cid = lax.axis_index("c")
    sid = lax.axis_index("s")
    wid = sid * _NC + cid
    pltpu.sync_copy(zeros_hbm.at[pl.ds(sid * rpt, rpt)],
                    acc.at[pl.ds(sid * rpt, rpt)])
    pltpu.sync_copy(ones_hbm, ones_v)
    pltpu.sync_copy(dstw.at[wid], didx_all)
    plsc.subcore_barrier()

    def group(j, carry):
      base = j * nb
      ds = [pltpu.async_copy(ones_v, acc.at[didx_all.at[base + b]], sems[b],
                             add=True) for b in range(nb)]
      for d_ in ds:
        d_.wait()
      return carry

    lax.fori_loop(0, c_chunks // nb, group, 0)
    plsc.subcore_barrier()
    pltpu.sync_copy(acc.at[pl.ds(sid * rpt, rpt)],
                    out_hbm.at[cid, pl.ds(sid * rpt, rpt)])

  return pl.kernel(
      body,
      out_type=jax.ShapeDtypeStruct((_NC, npad, w), jnp.float32),
      mesh=mesh,
      scratch_types=[
          pltpu.VMEM((c_chunks, _K), jnp.int32),
          pltpu.VMEM((_K, w), jnp.float32),
          pltpu.VMEM_SHARED((npad, w), jnp.float32),
      ] + [pltpu.SemaphoreType.DMA] * nb,
  )


def _agg_kernel(npad, c_chunks, d):
  rpt = npad // _NS
  mesh = plsc.VectorSubcoreMesh(core_axis_name="c", subcore_axis_name="s")

  nb = 2
  assert c_chunks % nb == 0

  def body(rows_hbm, srcw, dstw, zeros_hbm, out_hbm, sidx_all, didx_all,
           *bufs_and_sems):
    rows = bufs_and_sems[:nb]
    acc = bufs_and_sems[nb]
    gsem = bufs_and_sems[nb + 1:2 * nb + 1]
    ssem = bufs_and_sems[2 * nb + 1:]
    cid = lax.axis_index("c")
    sid = lax.axis_index("s")
    wid = sid * _NC + cid
    pltpu.sync_copy(zeros_hbm.at[pl.ds(sid * rpt, rpt)],
                    acc.at[pl.ds(sid * rpt, rpt)])
    pltpu.sync_copy(srcw.at[wid], sidx_all)
    pltpu.sync_copy(dstw.at[wid], didx_all)
    plsc.subcore_barrier()

    def group(j, carry):
      base = j * nb
      gd = [pltpu.async_copy(rows_hbm.at[sidx_all.at[base + b]], rows[b],
                             gsem[b]) for b in range(nb)]
      sd = []
      for b in range(nb):
        gd[b].wait()
        sd.append(pltpu.async_copy(rows[b], acc.at[didx_all.at[base + b]],
                                   ssem[b], add=True))
      for d_ in sd:
        d_.wait()
      return carry

    lax.fori_loop(0, c_chunks // nb, group, 0)
    plsc.subcore_barrier()
    pltpu.sync_copy(acc.at[pl.ds(sid * rpt, rpt)],
                    out_hbm.at[cid, pl.ds(sid * rpt, rpt)])

  return pl.kernel(
      body,
      out_type=jax.ShapeDtypeStruct((_NC, npad, d), jnp.float32),
      mesh=mesh,
      scratch_types=[
          pltpu.VMEM((c_chunks, _K), jnp.int32),
          pltpu.VMEM((c_chunks, _K), jnp.int32),
      ] + [pltpu.VMEM((_K, d), jnp.float32)] * nb + [
          pltpu.VMEM_SHARED((npad, d), jnp.float32),
      ] + [pltpu.SemaphoreType.DMA] * (2 * nb),
  )


def _scale_body(degp_ref, x_ref, y_ref, dinv_ref):
  dinv = lax.rsqrt(degp_ref[0, :, 0:1] + degp_ref[1, :, 0:1] + 1.0)
  y_ref[...] = x_ref[...] * dinv
  dinv_ref[...] = jnp.broadcast_to(dinv, dinv_ref.shape)


def _dense_body(dinv_ref, aggp_ref, y_ref, w1_ref, b1_ref, w2_ref, y2_ref):
  dinv = dinv_ref[:, 0:1]
  pre = (aggp_ref[0] + aggp_ref[1] + y_ref[...]) * dinv
  h = jnp.dot(pre.astype(jnp.bfloat16), w1_ref[...],
              preferred_element_type=jnp.float32)
  x1 = jnp.maximum(h + b1_ref[...], 0.0)
  h2 = jnp.dot(x1.astype(jnp.bfloat16), w2_ref[...],
               preferred_element_type=jnp.float32)
  y2_ref[...] = h2 * dinv


def _final_body(dinv_ref, aggp_ref, y2_ref, b2_ref, out_ref):
  dinv = dinv_ref[:, 0:1]
  o = (aggp_ref[0] + aggp_ref[1] + y2_ref[...]) * dinv + b2_ref[...]
  m = jnp.max(o, axis=1, keepdims=True)
  s = jnp.sum(jnp.exp(o - m), axis=1, keepdims=True)
  out_ref[...] = (o - m) - jnp.log(s)


def _row_specs(npad, d):
  dinv16 = pl.BlockSpec((_R, 16), lambda i: (i, 0))
  rows = pl.BlockSpec((_R, d), lambda i: (i, 0))
  aggp = pl.BlockSpec((2, _R, d), lambda i: (0, i, 0))
  return dinv16, rows, aggp


def kernel(x, edge_index, W1, b1, W2, b2):
  n, d_in = x.shape
  e = edge_index.shape[1]
  hid = W1.shape[1]
  d_out = W2.shape[1]

  npad = ((n + 1 + 255) // 256) * 256
  c_chunks = -(-e // (_W * _K))
  epad = _W * _K * c_chunks
  pad = epad - e

  x = x.astype(jnp.float32)
  src = jnp.concatenate([edge_index[0],
                         jnp.arange(pad, dtype=edge_index.dtype) % n])
  sink = n + (jnp.arange(pad, dtype=edge_index.dtype) % (npad - n))
  dst = jnp.concatenate([edge_index[1], sink])
  srcw = src.reshape(_W, c_chunks, _K)
  dstw = dst.reshape(_W, c_chunks, _K)
  x_p = jnp.pad(x, ((0, npad - n), (0, 0)))
  zeros_d = jnp.zeros((npad, d_in), jnp.float32)
  ones_d = jnp.ones((_K, d_in), jnp.float32)

  degp = _deg_kernel(npad, c_chunks, d_in)(dstw, zeros_d, ones_d)

  grid = (npad // _R,)
  dinv_s, row_s, aggp_s = _row_specs(npad, d_in)

  y, dinv16 = pl.pallas_call(
      _scale_body,
      grid=grid,
      in_specs=[pl.BlockSpec((2, _R, d_in), lambda i: (0, i, 0)), row_s],
      out_specs=[row_s, dinv_s],
      out_shape=[jax.ShapeDtypeStruct((npad, d_in), jnp.float32),
                 jax.ShapeDtypeStruct((npad, 16), jnp.float32)],
  )(degp, x_p)

  aggp1 = _agg_kernel(npad, c_chunks, d_in)(y, srcw, dstw, zeros_d)

  y2 = pl.pallas_call(
      _dense_body,
      grid=grid,
      in_specs=[
          dinv_s, aggp_s, row_s,
          pl.BlockSpec((d_in, hid), lambda i: (0, 0)),
          pl.BlockSpec((1, hid), lambda i: (0, 0)),
          pl.BlockSpec((hid, d_out), lambda i: (0, 0)),
      ],
      out_specs=pl.BlockSpec((_R, d_out), lambda i: (i, 0)),
      out_shape=jax.ShapeDtypeStruct((npad, d_out), jnp.float32),
  )(dinv16, aggp1, y, W1.astype(jnp.bfloat16), b1.reshape(1, hid),
    W2.astype(jnp.bfloat16))

  aggp2 = _agg_kernel(npad, c_chunks, d_out)(y2, srcw, dstw, zeros_d)

  dinv_s2, row_s2, aggp_s2 = _row_specs(npad, d_out)
  out = pl.pallas_call(
      _final_body,
      grid=grid,
      in_specs=[dinv_s2, aggp_s2, row_s2,
                pl.BlockSpec((1, d_out), lambda i: (0, 0))],
      out_specs=row_s2,
      out_shape=jax.ShapeDtypeStruct((npad, d_out), jnp.float32),
  )(dinv16, aggp2, y2, b2.reshape(1, d_out))

  return out[:n]


# speedup vs baseline: 1.0566x; 1.0566x over previous
import functools

import jax
import jax.numpy as jnp
from jax import lax
from jax.experimental import pallas as pl
from jax.experimental.pallas import tpu as pltpu
from jax.experimental.pallas import tpu_sc as plsc

_NC = 2
_NS = 16
_W = _NC * _NS
_K = 128
_R = 512


def _deg_kernel(npad, c_chunks, w=128):
  rpt = npad // _NS
  mesh = plsc.VectorSubcoreMesh(core_axis_name="c", subcore_axis_name="s")

  nb = 2
  assert c_chunks % nb == 0

  def body(dstw, zeros_hbm, ones_hbm, out_hbm, didx_all, ones_v, acc, *sems):
    cid = lax.axis_index("c")
    sid = lax.axis_index("s")
    wid = sid * _NC + cid
    pltpu.sync_copy(zeros_hbm.at[pl.ds(sid * rpt, rpt)],
                    acc.at[pl.ds(sid * rpt, rpt)])
    pltpu.sync_copy(ones_hbm, ones_v)
    pltpu.sync_copy(dstw.at[wid], didx_all)
    plsc.subcore_barrier()

    def swait(c, b):
      pltpu.make_async_copy(ones_v, acc.at[didx_all.at[c]], sems[b]).wait()

    def group(j, carry):
      base = j * nb
      for b in range(nb):
        c = base + b

        @pl.when(c >= nb)
        def _():
          swait(c - nb, b)

        pltpu.async_copy(ones_v, acc.at[didx_all.at[c]], sems[b], add=True)
      return carry

    lax.fori_loop(0, c_chunks // nb, group, 0)
    for b in range(nb):
      swait(c_chunks - nb + b, b)
    plsc.subcore_barrier()
    pltpu.sync_copy(acc.at[pl.ds(sid * rpt, rpt)],
                    out_hbm.at[cid, pl.ds(sid * rpt, rpt)])

  return pl.kernel(
      body,
      out_type=jax.ShapeDtypeStruct((_NC, npad, w), jnp.float32),
      mesh=mesh,
      scratch_types=[
          pltpu.VMEM((c_chunks, _K), jnp.int32),
          pltpu.VMEM((_K, w), jnp.float32),
          pltpu.VMEM_SHARED((npad, w), jnp.float32),
      ] + [pltpu.SemaphoreType.DMA] * nb,
  )


def _agg_kernel(npad, c_chunks, d, k):
  rpt = npad // _NS
  mesh = plsc.VectorSubcoreMesh(core_axis_name="c", subcore_axis_name="s")

  nbg = 2
  nbs = 1
  assert c_chunks % nbg == 0 and c_chunks > nbg, c_chunks

  def body(rows_hbm, srcw, dstw, zeros_hbm, out_hbm, sidx_all, didx_all,
           *bufs_and_sems):
    rows = bufs_and_sems[:nbg]
    acc = bufs_and_sems[nbg]
    gsem = bufs_and_sems[nbg + 1:nbg + 1 + nbg]
    ssem = bufs_and_sems[nbg + 1 + nbg:]
    cid = lax.axis_index("c")
    sid = lax.axis_index("s")
    wid = sid * _NC + cid

    def zpiece(p, carry):
      o = sid * rpt + p * 128
      pltpu.sync_copy(zeros_hbm.at[pl.ds(o, 128)], acc.at[pl.ds(o, 128)])
      return carry

    lax.fori_loop(0, rpt // 128, zpiece, 0)
    pltpu.sync_copy(srcw.at[wid], sidx_all)
    pltpu.sync_copy(dstw.at[wid], didx_all)
    plsc.subcore_barrier()

    def gissue(c, b):
      pltpu.async_copy(rows_hbm.at[sidx_all.at[c]], rows[b], gsem[b])

    def gwait(c, b):
      pltpu.make_async_copy(rows_hbm.at[sidx_all.at[c]], rows[b],
                            gsem[b]).wait()

    def swait(c, b):
      pltpu.make_async_copy(rows[b], acc.at[didx_all.at[c]],
                            ssem[b % nbs]).wait()

    for b in range(nbg):
      gissue(b, b)

    def group(j, carry):
      base = j * nbg
      for b in range(nbg):
        c = base + b
        gwait(c, b)

        @pl.when(c >= nbs)
        def _():
          swait(c - nbs, (b - nbs) % nbg)

        pltpu.async_copy(rows[b], acc.at[didx_all.at[c]], ssem[b % nbs],
                         add=True)

        @pl.when(jnp.logical_and(c >= nbs, c + nbg - nbs < c_chunks))
        def _():
          gissue(c + nbg - nbs, (b - nbs) % nbg)

      return carry

    lax.fori_loop(0, c_chunks // nbg, group, 0)
    for b in range(nbs):
      swait(c_chunks - nbs + b, (c_chunks - nbs + b) % nbg)
    plsc.subcore_barrier()

    def opiece(p, carry):
      o = sid * rpt + p * 128
      pltpu.sync_copy(acc.at[pl.ds(o, 128)], out_hbm.at[cid, pl.ds(o, 128)])
      return carry

    lax.fori_loop(0, rpt // 128, opiece, 0)

  return pl.kernel(
      body,
      out_type=jax.ShapeDtypeStruct((_NC, npad, d), jnp.float32),
      mesh=mesh,
      scratch_types=[
          pltpu.VMEM((c_chunks, k), jnp.int32),
          pltpu.VMEM((c_chunks, k), jnp.int32),
      ] + [pltpu.VMEM((k, d), jnp.float32)] * nbg + [
          pltpu.VMEM_SHARED((npad, d), jnp.float32),
      ] + [pltpu.SemaphoreType.DMA] * (nbg + nbs),
  )


def _scale_body(degp_ref, x_ref, y_ref, dinv_ref):
  dinv = lax.rsqrt(degp_ref[0, :, 0:1] + degp_ref[1, :, 0:1] + 1.0)
  y_ref[...] = x_ref[...] * dinv
  dinv_ref[...] = jnp.broadcast_to(dinv, dinv_ref.shape)


def _dense_body(dinv_ref, aggp_ref, y_ref, w1_ref, b1_ref, w2_ref, y2_ref):
  dinv = dinv_ref[:, 0:1]
  pre = (aggp_ref[0] + aggp_ref[1] + y_ref[...]) * dinv
  h = jnp.dot(pre.astype(jnp.bfloat16), w1_ref[...],
              preferred_element_type=jnp.float32)
  x1 = jnp.maximum(h + b1_ref[...], 0.0)
  h2 = jnp.dot(x1.astype(jnp.bfloat16), w2_ref[...],
               preferred_element_type=jnp.float32)
  y2_ref[...] = h2 * dinv


def _final_body(dinv_ref, aggp_ref, y2_ref, b2_ref, out_ref):
  dinv = dinv_ref[:, 0:1]
  o = (aggp_ref[0] + aggp_ref[1] + y2_ref[...]) * dinv + b2_ref[...]
  m = jnp.max(o, axis=1, keepdims=True)
  s = jnp.sum(jnp.exp(o - m), axis=1, keepdims=True)
  out_ref[...] = (o - m) - jnp.log(s)


def _row_specs(npad, d):
  dinv16 = pl.BlockSpec((_R, 16), lambda i: (i, 0))
  rows = pl.BlockSpec((_R, d), lambda i: (i, 0))
  aggp = pl.BlockSpec((2, _R, d), lambda i: (0, i, 0))
  return dinv16, rows, aggp


def kernel(x, edge_index, W1, b1, W2, b2):
  n, d_in = x.shape
  e = edge_index.shape[1]
  hid = W1.shape[1]
  d_out = W2.shape[1]

  npad = ((n + 1 + 255) // 256) * 256
  c_chunks = -(-e // (_W * _K))
  epad = _W * _K * c_chunks
  pad = epad - e

  x = x.astype(jnp.float32)
  src = jnp.concatenate([edge_index[0],
                         jnp.arange(pad, dtype=edge_index.dtype) % n])
  sink = n + (jnp.arange(pad, dtype=edge_index.dtype) % (npad - n))
  dst = jnp.concatenate([edge_index[1], sink])
  srcw = src.reshape(_W, c_chunks, _K)
  dstw = dst.reshape(_W, c_chunks, _K)
  x_p = jnp.pad(x, ((0, npad - n), (0, 0)))
  zeros_d = jnp.zeros((npad, d_in), jnp.float32)
  ones_d = jnp.ones((_K, d_in), jnp.float32)

  degp = _deg_kernel(npad, c_chunks, d_in)(dstw, zeros_d, ones_d)

  grid = (npad // _R,)
  dinv_s, row_s, aggp_s = _row_specs(npad, d_in)

  y, dinv16 = pl.pallas_call(
      _scale_body,
      grid=grid,
      in_specs=[pl.BlockSpec((2, _R, d_in), lambda i: (0, i, 0)), row_s],
      out_specs=[row_s, dinv_s],
      out_shape=[jax.ShapeDtypeStruct((npad, d_in), jnp.float32),
                 jax.ShapeDtypeStruct((npad, 16), jnp.float32)],
  )(degp, x_p)

  ka = 128
  ca = epad // (_W * ka)
  srcwa = src.reshape(_W, ca, ka)
  dstwa = dst.reshape(_W, ca, ka)

  aggp1 = _agg_kernel(npad, ca, d_in, ka)(y, srcwa, dstwa, zeros_d)

  y2 = pl.pallas_call(
      _dense_body,
      grid=grid,
      in_specs=[
          dinv_s, aggp_s, row_s,
          pl.BlockSpec((d_in, hid), lambda i: (0, 0)),
          pl.BlockSpec((1, hid), lambda i: (0, 0)),
          pl.BlockSpec((hid, d_out), lambda i: (0, 0)),
      ],
      out_specs=pl.BlockSpec((_R, d_out), lambda i: (i, 0)),
      out_shape=jax.ShapeDtypeStruct((npad, d_out), jnp.float32),
  )(dinv16, aggp1, y, W1.astype(jnp.bfloat16), b1.reshape(1, hid),
    W2.astype(jnp.bfloat16))

  aggp2 = _agg_kernel(npad, ca, d_out, ka)(y2, srcwa, dstwa, zeros_d)

  dinv_s2, row_s2, aggp_s2 = _row_specs(npad, d_out)
  out = pl.pallas_call(
      _final_body,
      grid=grid,
      in_specs=[dinv_s2, aggp_s2, row_s2,
                pl.BlockSpec((1, d_out), lambda i: (0, 0))],
      out_specs=row_s2,
      out_shape=jax.ShapeDtypeStruct((npad, d_out), jnp.float32),
  )(dinv16, aggp2, y2, b2.reshape(1, d_out))

  return out[:n]


# speedup vs baseline: 1.1191x; 1.0592x over previous
import functools

import jax
import jax.numpy as jnp
from jax import lax
from jax.experimental import pallas as pl
from jax.experimental.pallas import tpu as pltpu
from jax.experimental.pallas import tpu_sc as plsc

_NC = 2
_NS = 16
_W = _NC * _NS
_K = 128
_R = 1024


def _deg_kernel(npad, c_chunks, w=128):
  rpt = npad // _NS
  mesh = plsc.VectorSubcoreMesh(core_axis_name="c", subcore_axis_name="s")

  nb = 2
  assert c_chunks % nb == 0

  def body(dstw, zeros_hbm, ones_hbm, out_hbm, didx_all, ones_v, acc, *sems):
    cid = lax.axis_index("c")
    sid = lax.axis_index("s")
    wid = sid * _NC + cid
    pltpu.sync_copy(zeros_hbm.at[pl.ds(sid * rpt, rpt)],
                    acc.at[pl.ds(sid * rpt, rpt)])
    pltpu.sync_copy(ones_hbm, ones_v)
    pltpu.sync_copy(dstw.at[wid], didx_all)
    plsc.subcore_barrier()

    def swait(c, b):
      pltpu.make_async_copy(ones_v, acc.at[didx_all.at[c]], sems[b]).wait()

    def group(j, carry):
      base = j * nb
      for b in range(nb):
        c = base + b

        @pl.when(c >= nb)
        def _():
          swait(c - nb, b)

        pltpu.async_copy(ones_v, acc.at[didx_all.at[c]], sems[b], add=True)
      return carry

    lax.fori_loop(0, c_chunks // nb, group, 0)
    for b in range(nb):
      swait(c_chunks - nb + b, b)
    plsc.subcore_barrier()
    pltpu.sync_copy(acc.at[pl.ds(sid * rpt, rpt)],
                    out_hbm.at[cid, pl.ds(sid * rpt, rpt)])

  return pl.kernel(
      body,
      out_type=jax.ShapeDtypeStruct((_NC, npad, w), jnp.float32),
      mesh=mesh,
      scratch_types=[
          pltpu.VMEM((c_chunks, _K), jnp.int32),
          pltpu.VMEM((_K, w), jnp.float32),
          pltpu.VMEM_SHARED((npad, w), jnp.float32),
      ] + [pltpu.SemaphoreType.DMA] * nb,
  )


def _agg_kernel(npad, c_chunks, d, k):
  rpt = npad // _NS
  mesh = plsc.VectorSubcoreMesh(core_axis_name="c", subcore_axis_name="s")

  nbg = 2
  nbs = 1
  assert c_chunks % nbg == 0 and c_chunks > nbg, c_chunks

  def body(rows_hbm, srcw, dstw, zeros_hbm, out_hbm, sidx_all, didx_all,
           *bufs_and_sems):
    rows = bufs_and_sems[:nbg]
    acc = bufs_and_sems[nbg]
    gsem = bufs_and_sems[nbg + 1:nbg + 1 + nbg]
    ssem = bufs_and_sems[nbg + 1 + nbg:]
    cid = lax.axis_index("c")
    sid = lax.axis_index("s")
    wid = sid * _NC + cid

    def zpiece(p, carry):
      o = sid * rpt + p * 128
      pltpu.sync_copy(zeros_hbm.at[pl.ds(o, 128)], acc.at[pl.ds(o, 128)])
      return carry

    lax.fori_loop(0, rpt // 128, zpiece, 0)
    pltpu.sync_copy(srcw.at[wid], sidx_all)
    pltpu.sync_copy(dstw.at[wid], didx_all)
    plsc.subcore_barrier()

    def gissue(c, b):
      pltpu.async_copy(rows_hbm.at[sidx_all.at[c]], rows[b], gsem[b])

    def gwait(c, b):
      pltpu.make_async_copy(rows_hbm.at[sidx_all.at[c]], rows[b],
                            gsem[b]).wait()

    def swait(c, b):
      pltpu.make_async_copy(rows[b], acc.at[didx_all.at[c]],
                            ssem[b % nbs]).wait()

    for b in range(nbg):
      gissue(b, b)

    def group(j, carry):
      base = j * nbg
      for b in range(nbg):
        c = base + b
        gwait(c, b)

        @pl.when(c >= nbs)
        def _():
          swait(c - nbs, (b - nbs) % nbg)

        pltpu.async_copy(rows[b], acc.at[didx_all.at[c]], ssem[b % nbs],
                         add=True)

        @pl.when(jnp.logical_and(c >= nbs, c + nbg - nbs < c_chunks))
        def _():
          gissue(c + nbg - nbs, (b - nbs) % nbg)

      return carry

    lax.fori_loop(0, c_chunks // nbg, group, 0)
    for b in range(nbs):
      swait(c_chunks - nbs + b, (c_chunks - nbs + b) % nbg)
    plsc.subcore_barrier()

    def opiece(p, carry):
      o = sid * rpt + p * 128
      pltpu.sync_copy(acc.at[pl.ds(o, 128)], out_hbm.at[cid, pl.ds(o, 128)])
      return carry

    lax.fori_loop(0, rpt // 128, opiece, 0)

  return pl.kernel(
      body,
      out_type=jax.ShapeDtypeStruct((_NC, npad, d), jnp.float32),
      mesh=mesh,
      scratch_types=[
          pltpu.VMEM((c_chunks, k), jnp.int32),
          pltpu.VMEM((c_chunks, k), jnp.int32),
      ] + [pltpu.VMEM((k, d), jnp.float32)] * nbg + [
          pltpu.VMEM_SHARED((npad, d), jnp.float32),
      ] + [pltpu.SemaphoreType.DMA] * (nbg + nbs),
  )


def _scale_body(degp_ref, x_ref, y_ref, dinv_ref):
  dinv = lax.rsqrt(degp_ref[0, :, 0:1] + degp_ref[1, :, 0:1] + 1.0)
  y_ref[...] = x_ref[...] * dinv
  dinv_ref[...] = jnp.broadcast_to(dinv, dinv_ref.shape)


def _dense_body(dinv_ref, aggp_ref, y_ref, w1_ref, b1_ref, w2_ref, y2_ref):
  dinv = dinv_ref[:, 0:1]
  pre = (aggp_ref[0] + aggp_ref[1] + y_ref[...]) * dinv
  h = jnp.dot(pre.astype(jnp.bfloat16), w1_ref[...],
              preferred_element_type=jnp.float32)
  x1 = jnp.maximum(h + b1_ref[...], 0.0)
  h2 = jnp.dot(x1.astype(jnp.bfloat16), w2_ref[...],
               preferred_element_type=jnp.float32)
  y2_ref[...] = h2 * dinv


def _final_body(dinv_ref, aggp_ref, y2_ref, b2_ref, out_ref):
  dinv = dinv_ref[:, 0:1]
  o = (aggp_ref[0] + aggp_ref[1] + y2_ref[...]) * dinv + b2_ref[...]
  m = jnp.max(o, axis=1, keepdims=True)
  s = jnp.sum(jnp.exp(o - m), axis=1, keepdims=True)
  out_ref[...] = (o - m) - jnp.log(s)


def _row_specs(npad, d):
  dinv16 = pl.BlockSpec((_R, 16), lambda i: (i, 0))
  rows = pl.BlockSpec((_R, d), lambda i: (i, 0))
  aggp = pl.BlockSpec((2, _R, d), lambda i: (0, i, 0))
  return dinv16, rows, aggp


def kernel(x, edge_index, W1, b1, W2, b2):
  n, d_in = x.shape
  e = edge_index.shape[1]
  hid = W1.shape[1]
  d_out = W2.shape[1]

  npad = ((n + 1 + 255) // 256) * 256
  c_chunks = -(-e // (_W * _K))
  epad = _W * _K * c_chunks
  pad = epad - e

  x = x.astype(jnp.float32)
  src = jnp.concatenate([edge_index[0],
                         jnp.arange(pad, dtype=edge_index.dtype) % n])
  sink = n + (jnp.arange(pad, dtype=edge_index.dtype) % (npad - n))
  dst = jnp.concatenate([edge_index[1], sink])
  srcw = src.reshape(_W, c_chunks, _K)
  dstw = dst.reshape(_W, c_chunks, _K)
  x_p = jnp.pad(x, ((0, npad - n), (0, 0)))
  zeros_d = jnp.zeros((npad, d_in), jnp.float32)
  ones_d = jnp.ones((_K, d_in), jnp.float32)

  degp = _deg_kernel(npad, c_chunks, d_in)(dstw, zeros_d, ones_d)

  grid = (npad // _R,)
  dinv_s, row_s, aggp_s = _row_specs(npad, d_in)

  y, dinv16 = pl.pallas_call(
      _scale_body,
      grid=grid,
      in_specs=[pl.BlockSpec((2, _R, d_in), lambda i: (0, i, 0)), row_s],
      out_specs=[row_s, dinv_s],
      out_shape=[jax.ShapeDtypeStruct((npad, d_in), jnp.float32),
                 jax.ShapeDtypeStruct((npad, 16), jnp.float32)],
  )(degp, x_p)

  ka = 128
  ca = epad // (_W * ka)
  srcwa = src.reshape(_W, ca, ka)
  dstwa = dst.reshape(_W, ca, ka)

  aggp1 = _agg_kernel(npad, ca, d_in, ka)(y, srcwa, dstwa, zeros_d)

  y2 = pl.pallas_call(
      _dense_body,
      grid=grid,
      in_specs=[
          dinv_s, aggp_s, row_s,
          pl.BlockSpec((d_in, hid), lambda i: (0, 0)),
          pl.BlockSpec((1, hid), lambda i: (0, 0)),
          pl.BlockSpec((hid, d_out), lambda i: (0, 0)),
      ],
      out_specs=pl.BlockSpec((_R, d_out), lambda i: (i, 0)),
      out_shape=jax.ShapeDtypeStruct((npad, d_out), jnp.float32),
  )(dinv16, aggp1, y, W1.astype(jnp.bfloat16), b1.reshape(1, hid),
    W2.astype(jnp.bfloat16))

  aggp2 = _agg_kernel(npad, ca, d_out, ka)(y2, srcwa, dstwa, zeros_d)

  dinv_s2, row_s2, aggp_s2 = _row_specs(npad, d_out)
  out = pl.pallas_call(
      _final_body,
      grid=grid,
      in_specs=[dinv_s2, aggp_s2, row_s2,
                pl.BlockSpec((1, d_out), lambda i: (0, 0))],
      out_specs=row_s2,
      out_shape=jax.ShapeDtypeStruct((n, d_out), jnp.float32),
  )(dinv16, aggp2, y2, b2.reshape(1, d_out))

  return out
